# baseline (speedup 1.0000x reference)
import functools

import jax
import jax.numpy as jnp
from jax import lax
from jax.experimental import pallas as pl
from jax.experimental.pallas import tpu as pltpu
from jax.experimental.pallas import tpu_sc as plsc

VOCAB = 100000
D = 768
BATCH = 4
SEQ = 2048
LANES = 16
VECS = D // LANES

_info = plsc.get_sparse_core_info()
NC = _info.num_cores
NS = _info.num_subcores
NW = NC * NS
S_PER_W = SEQ // NW
CH = 32
NCHUNK = (BATCH * S_PER_W) // CH


def _emb_kernel(x_hbm, tgt_hbm, pos_hbm, out_hbm,
                idx_v, pos_v, rows0, rows1, psem, g0, g1, t0, t1):
    wid = lax.axis_index("s") * NC + lax.axis_index("c")
    base = wid * S_PER_W

    pos_cp = pltpu.async_copy(pos_hbm.at[pl.ds(base, S_PER_W)], pos_v, psem)
    for c in range(NCHUNK):
        b, h = divmod(c, 2)
        pltpu.sync_copy(x_hbm.at[b, pl.ds(base + h * CH, CH)], idx_v.at[c])
    pos_cp.wait()

    bufs = (rows0, rows1)
    gsems = (g0, g1)
    ssems = (t0, t1)

    def fire_gather(c):
        return pltpu.async_copy(tgt_hbm.at[idx_v.at[c]], bufs[c % 2], gsems[c % 2])

    def fire_scatter(c):
        b, h = divmod(c, 2)
        return pltpu.async_copy(bufs[c % 2], out_hbm.at[b, pl.ds(base + h * CH, CH)],
                                ssems[c % 2])

    gh = [None] * NCHUNK
    sh = [None] * NCHUNK
    gh[0] = fire_gather(0)
    for c in range(NCHUNK):
        gh[c].wait()
        if c + 1 < NCHUNK:
            if c >= 1:
                sh[c - 1].wait()
            gh[c + 1] = fire_gather(c + 1)

        h = c % 2
        buf = bufs[c % 2]

        def body(r, carry, h=h, buf=buf):
            for k in range(VECS):
                v = pos_v[h * CH + r, pl.ds(k * LANES, LANES)]
                plsc.addupdate(buf.at[r, pl.ds(k * LANES, LANES)], v)
            return carry

        lax.fori_loop(0, CH, body, 0)
        sh[c] = fire_scatter(c)

    sh[NCHUNK - 2].wait()
    sh[NCHUNK - 1].wait()


@jax.jit
def _emb(x, tgt_emb, pos_emb):
    mesh = plsc.VectorSubcoreMesh(core_axis_name="c", subcore_axis_name="s")
    f = functools.partial(
        pl.kernel,
        out_type=jax.ShapeDtypeStruct((BATCH, SEQ, D), jnp.float32),
        mesh=mesh,
        scratch_types=[
            pltpu.VMEM((NCHUNK, CH), jnp.int32),
            pltpu.VMEM((S_PER_W, D), jnp.float32),
            pltpu.VMEM((CH, D), jnp.float32),
            pltpu.VMEM((CH, D), jnp.float32),
            pltpu.SemaphoreType.DMA,
            pltpu.SemaphoreType.DMA,
            pltpu.SemaphoreType.DMA,
            pltpu.SemaphoreType.DMA,
            pltpu.SemaphoreType.DMA,
        ],
    )(_emb_kernel)
    return f(x, tgt_emb, pos_emb)


def kernel(x, tgt_emb, pos_emb):
    return _emb(x.astype(jnp.int32), tgt_emb, pos_emb)


# speedup vs baseline: 1.0816x; 1.0816x over previous
import functools

import jax
import jax.numpy as jnp
from jax import lax
from jax.experimental import pallas as pl
from jax.experimental.pallas import tpu as pltpu
from jax.experimental.pallas import tpu_sc as plsc

D = 768
BATCH = 4
SEQ = 2048
LANES = 16
VECS = D // LANES

_info = plsc.get_sparse_core_info()
NC = _info.num_cores
NS = _info.num_subcores
NW = NC * NS
S_PER_W = SEQ // NW
CH = 32
NCHUNK = (BATCH * S_PER_W) // CH
NBUF = 3


def _emb_kernel(x_hbm, tgt_hbm, pos_hbm, out_hbm,
                idx_v, pos_v, rows0, rows1, rows2,
                isem, psem, g0, g1, g2, t0, t1, t2):
    wid = lax.axis_index("s") * NC + lax.axis_index("c")
    base = wid * S_PER_W

    idx_cps = [
        pltpu.async_copy(x_hbm.at[b, pl.ds(base, S_PER_W)], idx_v.at[b], isem)
        for b in range(BATCH)
    ]
    pos_cp = pltpu.async_copy(pos_hbm.at[pl.ds(base, S_PER_W)], pos_v, psem)
    for cp in idx_cps:
        cp.wait()

    bufs = (rows0, rows1, rows2)
    gsems = (g0, g1, g2)
    ssems = (t0, t1, t2)

    def fire_gather(c):
        b, h = divmod(c, 2)
        return pltpu.async_copy(tgt_hbm.at[idx_v.at[b, pl.ds(h * CH, CH)]],
                                bufs[c % NBUF], gsems[c % NBUF])

    def fire_scatter(c):
        b, h = divmod(c, 2)
        return pltpu.async_copy(bufs[c % NBUF],
                                out_hbm.at[b, pl.ds(base + h * CH, CH)],
                                ssems[c % NBUF])

    gh = [None] * NCHUNK
    sh = [None] * NCHUNK
    gh[0] = fire_gather(0)
    gh[1] = fire_gather(1)
    pos_cp.wait()
    for c in range(NCHUNK):
        gh[c].wait()
        if c + 2 < NCHUNK:
            if c >= 1:
                sh[c - 1].wait()
            gh[c + 2] = fire_gather(c + 2)

        h = c % 2
        buf = bufs[c % NBUF]

        def body(r, carry, h=h, buf=buf):
            for k in range(VECS):
                v = pos_v[h * CH + r, pl.ds(k * LANES, LANES)]
                plsc.addupdate(buf.at[r, pl.ds(k * LANES, LANES)], v)
            return carry

        lax.fori_loop(0, CH, body, 0)
        sh[c] = fire_scatter(c)

    sh[NCHUNK - 3].wait()
    sh[NCHUNK - 2].wait()
    sh[NCHUNK - 1].wait()


@jax.jit
def _emb(x, tgt_emb, pos_emb):
    mesh = plsc.VectorSubcoreMesh(core_axis_name="c", subcore_axis_name="s")
    f = functools.partial(
        pl.kernel,
        out_type=jax.ShapeDtypeStruct((BATCH, SEQ, D), jnp.float32),
        mesh=mesh,
        scratch_types=[
            pltpu.VMEM((BATCH, S_PER_W), jnp.int32),
            pltpu.VMEM((S_PER_W, D), jnp.float32),
            pltpu.VMEM((CH, D), jnp.float32),
            pltpu.VMEM((CH, D), jnp.float32),
            pltpu.VMEM((CH, D), jnp.float32),
            pltpu.SemaphoreType.DMA,
            pltpu.SemaphoreType.DMA,
            pltpu.SemaphoreType.DMA,
            pltpu.SemaphoreType.DMA,
            pltpu.SemaphoreType.DMA,
            pltpu.SemaphoreType.DMA,
            pltpu.SemaphoreType.DMA,
            pltpu.SemaphoreType.DMA,
        ],
    )(_emb_kernel)
    return f(x, tgt_emb, pos_emb)


def kernel(x, tgt_emb, pos_emb):
    return _emb(x.astype(jnp.int32), tgt_emb, pos_emb)


# speedup vs baseline: 1.1973x; 1.1070x over previous
import functools

import jax
import jax.numpy as jnp
from jax import lax
from jax.experimental import pallas as pl
from jax.experimental.pallas import tpu as pltpu
from jax.experimental.pallas import tpu_sc as plsc

D = 768
BATCH = 4
SEQ = 2048
LANES = 16
VECS = D // LANES

_info = plsc.get_sparse_core_info()
NC = _info.num_cores
NS = _info.num_subcores
NW = NC * NS
S_PER_W = SEQ // NW
CH = 32
NCHUNK = (BATCH * S_PER_W) // CH
NBUF = 3


def _emb_kernel(x_hbm, tgt_hbm, pos_hbm, out_hbm,
                idx_v, pos_v, rows0, rows1, rows2,
                isem, psem, g0, g1, g2, t0, t1, t2):
    wid = lax.axis_index("s") * NC + lax.axis_index("c")
    base = wid * S_PER_W

    idx_cps = [
        pltpu.async_copy(x_hbm.at[b, pl.ds(base, S_PER_W)], idx_v.at[b], isem)
        for b in range(BATCH)
    ]
    pos_cp = pltpu.async_copy(pos_hbm.at[pl.ds(base, S_PER_W)], pos_v, psem)
    for cp in idx_cps:
        cp.wait()

    bufs = (rows0, rows1, rows2)
    gsems = (g0, g1, g2)
    ssems = (t0, t1, t2)

    def fire_gather(c):
        b, h = divmod(c, 2)
        return pltpu.async_copy(tgt_hbm.at[idx_v.at[b, pl.ds(h * CH, CH)]],
                                bufs[c % NBUF], gsems[c % NBUF])

    def fire_scatter(c):
        b, h = divmod(c, 2)
        return pltpu.async_copy(bufs[c % NBUF],
                                out_hbm.at[b, pl.ds(base + h * CH, CH)],
                                ssems[c % NBUF])

    gh = [None] * NCHUNK
    sh = [None] * NCHUNK
    gh[0] = fire_gather(0)
    gh[1] = fire_gather(1)
    pos_cp.wait()
    for c in range(NCHUNK):
        gh[c].wait()
        if c + 2 < NCHUNK:
            if c >= 1:
                sh[c - 1].wait()
            gh[c + 2] = fire_gather(c + 2)

        h = c % 2
        buf = bufs[c % NBUF]

        @plsc.parallel_loop(0, CH, 1, unroll=2)
        def _add(r, h=h, buf=buf):
            for k in range(VECS):
                v = pos_v[h * CH + r, pl.ds(k * LANES, LANES)]
                plsc.addupdate(buf.at[r, pl.ds(k * LANES, LANES)], v)

        sh[c] = fire_scatter(c)

    sh[NCHUNK - 3].wait()
    sh[NCHUNK - 2].wait()
    sh[NCHUNK - 1].wait()


@jax.jit
def _emb(x, tgt_emb, pos_emb):
    mesh = plsc.VectorSubcoreMesh(core_axis_name="c", subcore_axis_name="s")
    f = functools.partial(
        pl.kernel,
        out_type=jax.ShapeDtypeStruct((BATCH, SEQ, D), jnp.float32),
        mesh=mesh,
        scratch_types=[
            pltpu.VMEM((BATCH, S_PER_W), jnp.int32),
            pltpu.VMEM((S_PER_W, D), jnp.float32),
            pltpu.VMEM((CH, D), jnp.float32),
            pltpu.VMEM((CH, D), jnp.float32),
            pltpu.VMEM((CH, D), jnp.float32),
            pltpu.SemaphoreType.DMA,
            pltpu.SemaphoreType.DMA,
            pltpu.SemaphoreType.DMA,
            pltpu.SemaphoreType.DMA,
            pltpu.SemaphoreType.DMA,
            pltpu.SemaphoreType.DMA,
            pltpu.SemaphoreType.DMA,
            pltpu.SemaphoreType.DMA,
        ],
    )(_emb_kernel)
    return f(x, tgt_emb, pos_emb)


def kernel(x, tgt_emb, pos_emb):
    return _emb(x.astype(jnp.int32), tgt_emb, pos_emb)
